# baseline (speedup 1.0000x reference)
import functools

import jax
import jax.numpy as jnp
from jax import lax
from jax.experimental import pallas as pl
from jax.experimental.pallas import tpu as pltpu
from jax.experimental.pallas import tpu_sc as plsc

B = 16384
S = 200
D = 32
HID = 128
OUT = 64

NC = 2
NS = 16
NW = NC * NS
BPW = B // NW
G = 8
NG = BPW // G
IPG = G * S
NBUF = 4

SPLITS = ((0, 128), (128, S - 128))


def _pool_body(x_hbm, table_hbm, out_hbm, idx_v, rows_v, out_v, idx_sem, gat_sem):
    wid = lax.axis_index("s") * NC + lax.axis_index("c")
    base = wid * BPW

    def start_gathers(slot):
        for i in range(G):
            for off, sz in SPLITS:
                pltpu.async_copy(
                    table_hbm.at[idx_v.at[slot, i, pl.ds(off, sz)]],
                    rows_v.at[slot, pl.ds(i * S + off, sz)],
                    gat_sem,
                )

    def drain_gathers(slot):
        for i in range(G):
            for off, sz in SPLITS:
                pltpu.make_async_copy(
                    table_hbm.at[idx_v.at[slot, i, pl.ds(off, sz)]],
                    rows_v.at[slot, pl.ds(i * S + off, sz)],
                    gat_sem,
                ).wait()

    def start_idx(g, slot):
        pltpu.async_copy(x_hbm.at[pl.ds(base + g * G, G)], idx_v.at[slot], idx_sem)

    def wait_idx(slot):
        pltpu.make_async_copy(x_hbm.at[pl.ds(0, G)], idx_v.at[slot], idx_sem).wait()

    for k in range(NBUF):
        start_idx(k, k)
    for k in range(NBUF - 1):
        wait_idx(k)
        start_gathers(k)

    def group(g, carry):
        buf = g % NBUF
        drain_gathers(buf)

        @pl.when(g + NBUF < NG)
        def _():
            gg = jnp.minimum(g + NBUF, NG - 1)
            start_idx(gg, buf)

        @pl.when(g + NBUF - 1 < NG)
        def _():
            nxt = (g + NBUF - 1) % NBUF
            wait_idx(nxt)
            start_gathers(nxt)

        for i in range(G):
            def rbody(r, accs, i=i):
                a0, a1 = accs
                for u in range(10):
                    rr = i * S + r * 10 + u
                    v = plsc.bitcast(rows_v[buf, rr, 0:32], jnp.int32)
                    a0 = a0 + plsc.bitcast(v << 16, jnp.float32)
                    a1 = a1 + plsc.bitcast(v & jnp.int32(-65536), jnp.float32)
                return a0, a1

            acc0, acc1 = lax.fori_loop(
                0, 20, rbody, (jnp.zeros((16,), jnp.float32), jnp.zeros((16,), jnp.float32))
            )
            b_loc = g * G + i
            out_v[b_loc, 0:16] = acc0
            out_v[b_loc, 16:32] = acc1
        return carry

    lax.fori_loop(0, NG, group, 0)
    pltpu.sync_copy(out_v, out_hbm.at[pl.ds(base, BPW)])


@jax.jit
def _pool(xf, table):
    mesh = plsc.VectorSubcoreMesh(core_axis_name="c", subcore_axis_name="s")
    return pl.kernel(
        _pool_body,
        out_type=jax.ShapeDtypeStruct((B, D), jnp.float32),
        mesh=mesh,
        scratch_types=[
            pltpu.VMEM((NBUF, G, S), jnp.int32),
            pltpu.VMEM((NBUF, IPG, D), jnp.bfloat16),
            pltpu.VMEM((BPW, D), jnp.float32),
            pltpu.SemaphoreType.DMA,
            pltpu.SemaphoreType.DMA,
        ],
        compiler_params=pltpu.CompilerParams(
            use_tc_tiling_on_sc=False, needs_layout_passes=False
        ),
    )(xf, table)


def _mlp_body(p_ref, w1_ref, b1_ref, w2_ref, b2_ref, o_ref):
    h = jnp.dot(p_ref[...], w1_ref[...], preferred_element_type=jnp.float32)
    h = jnp.maximum(h + b1_ref[...], 0.0)
    o = jnp.dot(h, w2_ref[...], preferred_element_type=jnp.float32)
    o_ref[...] = o + b2_ref[...]


@jax.jit
def _mlp(pooled, W1t, b1, W2t, b2):
    BT = 1024
    return pl.pallas_call(
        _mlp_body,
        grid=(B // BT,),
        in_specs=[
            pl.BlockSpec((BT, D), lambda i: (i, 0)),
            pl.BlockSpec((D, HID), lambda i: (0, 0)),
            pl.BlockSpec((1, HID), lambda i: (0, 0)),
            pl.BlockSpec((HID, OUT), lambda i: (0, 0)),
            pl.BlockSpec((1, OUT), lambda i: (0, 0)),
        ],
        out_specs=pl.BlockSpec((BT, OUT), lambda i: (i, 0)),
        out_shape=jax.ShapeDtypeStruct((B, OUT), jnp.float32),
    )(pooled, W1t, b1, W2t, b2)


def kernel(x, table, W1, b1, W2, b2):
    sums = _pool(x.astype(jnp.int32), table.astype(jnp.bfloat16))
    perm = jnp.concatenate([jnp.arange(0, D, 2), jnp.arange(1, D, 2)])
    W1t = W1.T[perm, :] * jnp.float32(1.0 / S)
    return _mlp(sums, W1t, b1.reshape(1, HID), W2.T, b2.reshape(1, OUT))


# speedup vs baseline: 1.0252x; 1.0252x over previous
import functools

import jax
import jax.numpy as jnp
from jax import lax
from jax.experimental import pallas as pl
from jax.experimental.pallas import tpu as pltpu
from jax.experimental.pallas import tpu_sc as plsc

B = 16384
S = 200
D = 32
HID = 128
OUT = 64

NC = 2
NS = 16
NW = NC * NS
BPW = B // NW
G = 8
NG = BPW // G
IPG = G * S
NBUF = 4

TAIL = S - 128


def _pool_body(x_hbm, table_hbm, out_hbm, idx_v, rows_v, out_v, idx_sem, gat_sem):
    wid = lax.axis_index("s") * NC + lax.axis_index("c")
    base = wid * BPW

    def start_gathers(slot):
        for i in range(G):
            pltpu.async_copy(
                table_hbm.at[idx_v.at[slot, 0, i]],
                rows_v.at[slot, pl.ds(i * S, 128)],
                gat_sem,
            )
            pltpu.async_copy(
                table_hbm.at[idx_v.at[slot, 1, i, pl.ds(0, TAIL)]],
                rows_v.at[slot, pl.ds(i * S + 128, TAIL)],
                gat_sem,
            )

    def drain_gathers(slot):
        for i in range(G):
            pltpu.make_async_copy(
                table_hbm.at[idx_v.at[slot, 0, i]],
                rows_v.at[slot, pl.ds(i * S, 128)],
                gat_sem,
            ).wait()
            pltpu.make_async_copy(
                table_hbm.at[idx_v.at[slot, 1, i, pl.ds(0, TAIL)]],
                rows_v.at[slot, pl.ds(i * S + 128, TAIL)],
                gat_sem,
            ).wait()

    def start_idx(g, slot):
        b0 = base + g * G
        pltpu.async_copy(x_hbm.at[0, pl.ds(b0, G)], idx_v.at[slot, 0], idx_sem)
        pltpu.async_copy(x_hbm.at[1, pl.ds(b0, G)], idx_v.at[slot, 1], idx_sem)

    def wait_idx(slot):
        for h in range(2):
            pltpu.make_async_copy(
                x_hbm.at[0, pl.ds(0, G)], idx_v.at[slot, h], idx_sem
            ).wait()

    for k in range(NBUF):
        start_idx(k, k)
    for k in range(NBUF - 1):
        wait_idx(k)
        start_gathers(k)

    def group(g, carry):
        buf = g % NBUF
        drain_gathers(buf)

        @pl.when(g + NBUF < NG)
        def _():
            gg = jnp.minimum(g + NBUF, NG - 1)
            start_idx(gg, buf)

        @pl.when(g + NBUF - 1 < NG)
        def _():
            nxt = (g + NBUF - 1) % NBUF
            wait_idx(nxt)
            start_gathers(nxt)

        for i in range(G):
            def rbody(r, accs, i=i):
                a0, a1 = accs
                for u in range(10):
                    rr = i * S + r * 10 + u
                    v = plsc.bitcast(rows_v[buf, rr, 0:32], jnp.int32)
                    a0 = a0 + plsc.bitcast(v << 16, jnp.float32)
                    a1 = a1 + plsc.bitcast(v & jnp.int32(-65536), jnp.float32)
                return a0, a1

            acc0, acc1 = lax.fori_loop(
                0, 20, rbody, (jnp.zeros((16,), jnp.float32), jnp.zeros((16,), jnp.float32))
            )
            b_loc = g * G + i
            out_v[b_loc, 0:16] = acc0
            out_v[b_loc, 16:32] = acc1
        return carry

    lax.fori_loop(0, NG, group, 0)
    pltpu.sync_copy(out_v, out_hbm.at[pl.ds(base, BPW)])


@jax.jit
def _pool(xf, table):
    mesh = plsc.VectorSubcoreMesh(core_axis_name="c", subcore_axis_name="s")
    return pl.kernel(
        _pool_body,
        out_type=jax.ShapeDtypeStruct((B, D), jnp.float32),
        mesh=mesh,
        scratch_types=[
            pltpu.VMEM((NBUF, 2, G, 128), jnp.int32),
            pltpu.VMEM((NBUF, IPG, D), jnp.bfloat16),
            pltpu.VMEM((BPW, D), jnp.float32),
            pltpu.SemaphoreType.DMA,
            pltpu.SemaphoreType.DMA,
        ],
        compiler_params=pltpu.CompilerParams(
            use_tc_tiling_on_sc=False, needs_layout_passes=False
        ),
    )(xf, table)


CH = 32
NCH = BPW // CH


def _detile_body(x_hbm, out_hbm, v, sem):
    wid = lax.axis_index("s") * NC + lax.axis_index("c")
    b0w = wid * BPW

    def step(k, carry):
        buf = k % 2
        b0 = b0w + k * CH

        @pl.when(k >= 2)
        def _():
            for h in range(2):
                pltpu.make_async_copy(
                    v.at[buf, h], out_hbm.at[h, pl.ds(b0w, CH)], sem
                ).wait()

        for h in range(2):
            pltpu.sync_copy(
                x_hbm.at[pl.ds(b0, CH), pl.ds(128 * h, 128)], v.at[buf, h]
            )
        for h in range(2):
            pltpu.async_copy(v.at[buf, h], out_hbm.at[h, pl.ds(b0, CH)], sem)
        return carry

    lax.fori_loop(0, NCH, step, 0)
    for buf in range(2):
        for h in range(2):
            pltpu.make_async_copy(
                v.at[buf, h], out_hbm.at[h, pl.ds(b0w, CH)], sem
            ).wait()


@jax.jit
def _detile(x):
    mesh = plsc.VectorSubcoreMesh(core_axis_name="c", subcore_axis_name="s")
    return pl.kernel(
        _detile_body,
        out_type=jax.ShapeDtypeStruct((2, B, 128), jnp.int32),
        mesh=mesh,
        scratch_types=[
            pltpu.VMEM((2, 2, CH, 128), jnp.int32),
            pltpu.SemaphoreType.DMA,
        ],
        compiler_params=pltpu.CompilerParams(use_tc_tiling_on_sc=True),
    )(x)


def _mlp_body(p_ref, w1_ref, b1_ref, w2_ref, b2_ref, o_ref):
    h = jnp.dot(p_ref[...], w1_ref[...], preferred_element_type=jnp.float32)
    h = jnp.maximum(h + b1_ref[...], 0.0)
    o = jnp.dot(h, w2_ref[...], preferred_element_type=jnp.float32)
    o_ref[...] = o + b2_ref[...]


@jax.jit
def _mlp(pooled, W1t, b1, W2t, b2):
    BT = 1024
    return pl.pallas_call(
        _mlp_body,
        grid=(B // BT,),
        in_specs=[
            pl.BlockSpec((BT, D), lambda i: (i, 0)),
            pl.BlockSpec((D, HID), lambda i: (0, 0)),
            pl.BlockSpec((1, HID), lambda i: (0, 0)),
            pl.BlockSpec((HID, OUT), lambda i: (0, 0)),
            pl.BlockSpec((1, OUT), lambda i: (0, 0)),
        ],
        out_specs=pl.BlockSpec((BT, OUT), lambda i: (i, 0)),
        out_shape=jax.ShapeDtypeStruct((B, OUT), jnp.float32),
    )(pooled, W1t, b1, W2t, b2)


def kernel(x, table, W1, b1, W2, b2):
    xp = jnp.pad(x.astype(jnp.int32), ((0, 0), (0, 256 - S)))
    sums = _pool(_detile(xp), table.astype(jnp.bfloat16))
    perm = jnp.concatenate([jnp.arange(0, D, 2), jnp.arange(1, D, 2)])
    W1t = W1.T[perm, :] * jnp.float32(1.0 / S)
    return _mlp(sums, W1t, b1.reshape(1, HID), W2.T, b2.reshape(1, OUT))
